# baseline (speedup 1.0000x reference)
import functools

import numpy as np
import jax
import jax.numpy as jnp
from jax import lax
from jax.experimental import pallas as pl
from jax.experimental.pallas import tpu as pltpu
from jax.experimental.pallas import tpu_sc as plsc

_E = 8
_L = 16

_SCHED = 1.0 - np.exp(-1.0 / 1.0)
_K = max(int(_E - np.floor(_SCHED * _E)), 1)


def _gate_body(ew_hbm, bias_hbm, permt_hbm, out_hbm, ew_v, bias_v, permt_v, out_v):
    wid = lax.axis_index("s") * 2 + lax.axis_index("c")

    @pl.when(wid == 0)
    def _():
        pltpu.sync_copy(ew_hbm, ew_v)
        pltpu.sync_copy(bias_hbm, bias_v)
        pltpu.sync_copy(permt_hbm, permt_v)

        lvv = ew_v[...] + bias_v[...]
        lv = [lvv[j] for j in range(_E)]

        one, zero = jnp.int32(1), jnp.int32(0)
        sel = []
        for j in range(_E):
            rank = zero
            for i in range(_E):
                if i == j:
                    continue
                ahead = lv[i] > lv[j]
                if i < j:
                    ahead = ahead | (lv[i] == lv[j])
                rank = rank + jnp.where(ahead, one, zero)
            sel.append(rank < _K)

        xs = [jnp.where(sel[j], lv[j], jnp.float32(-1e9)) for j in range(_E)]
        m = xs[0]
        for j in range(1, _E):
            m = jnp.maximum(m, xs[j])

        lanes = lax.iota(jnp.int32, _L)
        xv = jnp.full((_L,), -1e30, jnp.float32)
        for j in range(_E):
            xv = jnp.where(lanes == j, xs[j], xv)
        ev = jnp.exp(xv - m)

        s = ev[0]
        for j in range(1, _E):
            s = s + ev[j]

        acc = ev[0] * permt_v[0, :]
        for j in range(1, _E):
            acc = acc + ev[j] * permt_v[j, :]

        out_v[...] = acc / s
        pltpu.sync_copy(out_v, out_hbm)


@jax.jit
def _gate(ew16, bias16, permt):
    mesh = plsc.VectorSubcoreMesh(core_axis_name="c", subcore_axis_name="s")
    return pl.kernel(
        _gate_body,
        out_type=jax.ShapeDtypeStruct((_L,), jnp.float32),
        mesh=mesh,
        scratch_types=[
            pltpu.VMEM((_L,), jnp.float32),
            pltpu.VMEM((_L,), jnp.float32),
            pltpu.VMEM((_E, _L), jnp.float32),
            pltpu.VMEM((_L,), jnp.float32),
        ],
    )(ew16, bias16, permt)


_T1 = 1536
_C = 4
_NW = 32


def _sc_combine_body(probs_hbm, h_hbm, out_hbm, probs_v, bufs, obuf, ldsem, stsem):
    wid = lax.axis_index("s") * 2 + lax.axis_index("c")
    rpw = _T1 // _NW
    nch = rpw // _C
    D = h_hbm.shape[2]

    pltpu.sync_copy(probs_hbm, probs_v)
    pv = probs_v[...]
    ps = [pv[e] for e in range(_E)]
    base = wid * rpw

    ld = {}
    st = {}

    def start_loads(ch):
        sl = ch % 2
        row = base + ch * _C
        ld[ch] = [
            pltpu.async_copy(h_hbm.at[e, pl.ds(row, _C)], bufs.at[sl, e], ldsem.at[sl])
            for e in range(_E)
        ]

    start_loads(0)
    if nch > 1:
        start_loads(1)
    for ch in range(nch):
        sl = ch % 2
        for d in ld.pop(ch):
            d.wait()
        if ch >= 2:
            st.pop(ch - 2).wait()

        for r in range(_C):
            def body(i, carry, r=r):
                c = pl.multiple_of(i * _L, _L)
                acc = ps[0] * bufs[sl, 0, r, pl.ds(c, _L)]
                for e in range(1, _E):
                    acc = acc + ps[e] * bufs[sl, e, r, pl.ds(c, _L)]
                obuf[sl, r, pl.ds(c, _L)] = acc
                return carry
            lax.fori_loop(0, D // _L, body, 0)

        row = base + ch * _C
        st[ch] = pltpu.async_copy(obuf.at[sl], out_hbm.at[pl.ds(row, _C)], stsem.at[sl])
        if ch + 2 < nch:
            start_loads(ch + 2)
    for d in st.values():
        d.wait()


@jax.jit
def _sc_combine(probs16, h):
    _, _, D = h.shape
    mesh = plsc.VectorSubcoreMesh(core_axis_name="c", subcore_axis_name="s")
    return pl.kernel(
        _sc_combine_body,
        out_type=jax.ShapeDtypeStruct((_T1, D), jnp.float32),
        mesh=mesh,
        scratch_types=[
            pltpu.VMEM((_L,), jnp.float32),
            pltpu.VMEM((2, _E, _C, D), jnp.float32),
            pltpu.VMEM((2, _C, D), jnp.float32),
            pltpu.SemaphoreType.DMA((2,)),
            pltpu.SemaphoreType.DMA((2,)),
        ],
    )(probs16, h)


def _combine_body(probs_ref, h_ref, o_ref):
    acc = probs_ref[0] * h_ref[0]
    for e in range(1, _E):
        acc = acc + probs_ref[e] * h_ref[e]
    o_ref[...] = acc


@jax.jit
def _combine(probs16, h):
    E, T, D = h.shape
    tt = 256
    skip = _T1 // tt
    return pl.pallas_call(
        _combine_body,
        grid=(T // tt - skip,),
        in_specs=[
            pl.BlockSpec(memory_space=pltpu.SMEM),
            pl.BlockSpec((E, tt, D), lambda i: (0, i + skip, 0)),
        ],
        out_specs=pl.BlockSpec((tt, D), lambda i: (i + skip, 0)),
        out_shape=jax.ShapeDtypeStruct((T, D), jnp.float32),
        compiler_params=pltpu.CompilerParams(
            dimension_semantics=("arbitrary",),
        ),
    )(probs16, h)


def kernel(h, x, permutation_weights, expert_weights, bias):
    del x
    ew16 = jnp.pad(expert_weights[:, 0], (0, _L - _E))
    bias16 = jnp.pad(bias, (0, _L - _E))
    permt = jnp.pad(permutation_weights.T, ((0, 0), (0, _L - _E)))
    probs16 = _gate(ew16, bias16, permt)
    y_tc = _combine(probs16, h)
    y_sc = _sc_combine(probs16, h)
    return lax.dynamic_update_slice(y_tc, y_sc, (0, 0))


# speedup vs baseline: 1.1850x; 1.1850x over previous
import numpy as np
import jax
import jax.numpy as jnp
from jax import lax
from jax.experimental import pallas as pl
from jax.experimental.pallas import tpu as pltpu
from jax.experimental.pallas import tpu_sc as plsc

_E = 8
_L = 16

_SCHED = 1.0 - np.exp(-1.0 / 1.0)
_K = max(int(_E - np.floor(_SCHED * _E)), 1)


def _gate_body(pk_hbm, p4_hbm, out_hbm, pk_v, p4_v, out_v, sems):
    wid = lax.axis_index("s") * 2 + lax.axis_index("c")

    @pl.when(wid == 0)
    def _():
        c1 = pltpu.async_copy(pk_hbm, pk_v, sems.at[0])
        c2 = pltpu.async_copy(p4_hbm, p4_v, sems.at[1])
        c1.wait()
        c2.wait()

        pk = pk_v[...]
        lv = [pk[j] + pk[j + _E] for j in range(_E)]

        one, zero = jnp.int32(1), jnp.int32(0)
        sel = []
        for j in range(_E):
            rank = zero
            for i in range(_E):
                if i == j:
                    continue
                ahead = lv[i] > lv[j]
                if i < j:
                    ahead = ahead | (lv[i] == lv[j])
                rank = rank + jnp.where(ahead, one, zero)
            sel.append(rank < _K)

        xs = [jnp.where(sel[j], lv[j], jnp.float32(-1e9)) for j in range(_E)]
        m = xs[0]
        for j in range(1, _E):
            m = jnp.maximum(m, xs[j])

        lanes = lax.iota(jnp.int32, _L)
        xv = jnp.full((_L,), -1e30, jnp.float32)
        for j in range(_E):
            xv = jnp.where(lanes == j, xs[j], xv)
        ev = jnp.exp(xv - m)
        es = [ev[j] for j in range(_E)]
        s = es[0]
        for j in range(1, _E):
            s = s + es[j]

        lanes8 = lanes & jnp.int32(_E - 1)
        q = jnp.zeros((_L,), jnp.float32)
        for j in range(_E):
            q = jnp.where(lanes8 == j, es[j], q)

        outs = []
        for k in range(_E // 2):
            w = p4_v[k, :] * q
            lo = w[0]
            hi = w[_E]
            for l in range(1, _E):
                lo = lo + w[l]
                hi = hi + w[l + _E]
            outs += [lo, hi]

        ov = jnp.zeros((_L,), jnp.float32)
        for i in range(_E):
            ov = jnp.where(lanes == i, outs[i], ov)
        out_v[...] = ov / s
        pltpu.sync_copy(out_v, out_hbm)


@jax.jit
def _gate(packed16, p4):
    mesh = plsc.VectorSubcoreMesh(core_axis_name="c", subcore_axis_name="s")
    return pl.kernel(
        _gate_body,
        out_type=jax.ShapeDtypeStruct((_L,), jnp.float32),
        mesh=mesh,
        scratch_types=[
            pltpu.VMEM((_L,), jnp.float32),
            pltpu.VMEM((_E // 2, _L), jnp.float32),
            pltpu.VMEM((_L,), jnp.float32),
            pltpu.SemaphoreType.DMA((2,)),
        ],
    )(packed16, p4)


def _combine_body(probs_ref, h_ref, o_ref):
    acc = probs_ref[0] * h_ref[0]
    for e in range(1, _E):
        acc = acc + probs_ref[e] * h_ref[e]
    o_ref[...] = acc


@jax.jit
def _combine(probs16, h):
    E, T, D = h.shape
    tt = 256
    return pl.pallas_call(
        _combine_body,
        grid=(T // tt,),
        in_specs=[
            pl.BlockSpec(memory_space=pltpu.SMEM),
            pl.BlockSpec((E, tt, D), lambda i: (0, i, 0)),
        ],
        out_specs=pl.BlockSpec((tt, D), lambda i: (i, 0)),
        out_shape=jax.ShapeDtypeStruct((T, D), jnp.float32),
        compiler_params=pltpu.CompilerParams(
            dimension_semantics=("arbitrary",),
        ),
    )(probs16, h)


def kernel(h, x, permutation_weights, expert_weights, bias):
    del x
    packed16 = jnp.concatenate([expert_weights[:, 0], bias])
    p4 = permutation_weights.reshape(_E // 2, _L)
    probs16 = _gate(packed16, p4)
    return _combine(probs16, h)
